# Initial kernel scaffold; baseline (speedup 1.0000x reference)
import jax
import jax.numpy as jnp
from jax.experimental import pallas as pl


def kernel(x, edge_index, W1, b1, W2, b2, Wfc, bfc):
    raise NotImplementedError("write your pallas kernel here")


# speedup vs baseline: 9.4324x; 9.4324x over previous
import jax
import jax.numpy as jnp
from jax import lax
from jax.experimental import pallas as pl
from jax.experimental.pallas import tpu as pltpu
from jax.experimental.pallas import tpu_sc as plsc

N = 10000
E = 320000
F = 128
H = 64
NC, NS, L = 2, 16, 16
CH = 128
EPAD = 323584
TRASH = N
ACC_ROWS = 10240
DEG_W = 80896
DEG_OUT = 10112

_mesh = plsc.VectorSubcoreMesh(
    core_axis_name="c", subcore_axis_name="s", num_cores=NC, num_subcores=NS)


def _deg_body(dst_hbm, out0_hbm, out1_hbm, deg_sh, zb_v, ones_v, dst_v):
    c = lax.axis_index("c")
    s = lax.axis_index("s")

    def _zb(k, carry):
        zb_v[pl.ds(k * L, L)] = jnp.zeros((L,), jnp.float32)
        return carry
    lax.fori_loop(0, DEG_OUT // L, _zb, 0)

    def _ob(k, carry):
        ones_v[pl.ds(k * L, L)] = jnp.ones((L,), jnp.float32)
        return carry
    lax.fori_loop(0, CH // L, _ob, 0)
    pltpu.sync_copy(zb_v.at[pl.ds(0, DEG_W // NS)],
                    deg_sh.at[pl.ds(s * (DEG_W // NS), DEG_W // NS)])
    plsc.subcore_barrier()

    e0 = c * (EPAD // NC) + s * (EPAD // (NC * NS))

    def _chunk(i, carry):
        pltpu.sync_copy(dst_hbm.at[pl.ds(e0 + i * CH, CH)], dst_v)
        pltpu.sync_copy(ones_v, deg_sh.at[dst_v], add=True)
        return carry
    lax.fori_loop(0, EPAD // (NC * NS) // CH, _chunk, 0)
    plsc.subcore_barrier()

    @pl.when(s == 0)
    def _():
        pltpu.sync_copy(deg_sh.at[pl.ds(0, DEG_OUT)], zb_v)

        @pl.when(c == 0)
        def _():
            pltpu.sync_copy(zb_v, out0_hbm)

        @pl.when(c == 1)
        def _():
            pltpu.sync_copy(zb_v, out1_hbm)


_deg_call = pl.kernel(
    _deg_body,
    out_type=(jax.ShapeDtypeStruct((DEG_OUT,), jnp.float32),
              jax.ShapeDtypeStruct((DEG_OUT,), jnp.float32)),
    mesh=_mesh,
    scratch_types=[
        pltpu.VMEM_SHARED((DEG_W,), jnp.float32),
        pltpu.VMEM((DEG_OUT,), jnp.float32),
        pltpu.VMEM((CH,), jnp.float32),
        pltpu.VMEM((CH,), jnp.int32),
    ],
)


def _scatter_body(h2_hbm, src_hbm, dst_hbm, out_hbm,
                  acc_sh, idx_v, srcb_v, dst_v, rows_v, orow_v):
    c = lax.axis_index("c")
    s = lax.axis_index("s")

    r0 = s * (ACC_ROWS // NS)

    def _init(i, carry):
        base = r0 + i * CH
        for j in range(CH // L):
            v = base + j * L + lax.iota(jnp.int32, L)
            v = jnp.minimum(v, N - 1)
            idx_v[pl.ds(j * L, L)] = v * 2 + c
        pltpu.sync_copy(h2_hbm.at[idx_v], rows_v)
        pltpu.sync_copy(rows_v, acc_sh.at[pl.ds(base, CH)])
        return carry
    lax.fori_loop(0, (ACC_ROWS // NS) // CH, _init, 0)
    plsc.subcore_barrier()

    e0 = s * (EPAD // NS)

    def _edge(i, carry):
        base = e0 + i * CH
        pltpu.sync_copy(src_hbm.at[pl.ds(base, CH)], srcb_v)
        pltpu.sync_copy(dst_hbm.at[pl.ds(base, CH)], dst_v)
        for j in range(CH // L):
            idx_v[pl.ds(j * L, L)] = srcb_v[pl.ds(j * L, L)] * 2 + c
        pltpu.sync_copy(h2_hbm.at[idx_v], rows_v)
        pltpu.sync_copy(rows_v, acc_sh.at[dst_v], add=True)
        return carry
    lax.fori_loop(0, EPAD // NS // CH, _edge, 0)
    plsc.subcore_barrier()

    QF = 632
    QL = N - 15 * QF
    w0 = s * QF

    @pl.when(s < 15)
    def _():
        pltpu.sync_copy(acc_sh.at[pl.ds(w0, QF)], orow_v)
        pltpu.sync_copy(orow_v, out_hbm.at[c, pl.ds(w0, QF)])

    @pl.when(s == 15)
    def _():
        pltpu.sync_copy(acc_sh.at[pl.ds(w0, QL)], orow_v.at[pl.ds(0, QL)])
        pltpu.sync_copy(orow_v.at[pl.ds(0, QL)], out_hbm.at[c, pl.ds(w0, QL)])


_scatter_call = pl.kernel(
    _scatter_body,
    out_type=jax.ShapeDtypeStruct((NC, N, H), jnp.float32),
    mesh=_mesh,
    compiler_params=pltpu.CompilerParams(use_tc_tiling_on_sc=False),
    scratch_types=[
        pltpu.VMEM_SHARED((ACC_ROWS, H), jnp.float32),
        pltpu.VMEM((CH,), jnp.int32),
        pltpu.VMEM((CH,), jnp.int32),
        pltpu.VMEM((CH,), jnp.int32),
        pltpu.VMEM((CH, H), jnp.float32),
        pltpu.VMEM((632, H), jnp.float32),
    ],
)


def _dis_body(d0_ref, d1_ref, dis_ref):
    deg = d0_ref[...] + d1_ref[...] + 1.0
    dis_ref[...] = lax.rsqrt(deg)


def _mm_scale_body(x_ref, wt_ref, dis_ref, o_ref):
    h = jnp.dot(x_ref[...], wt_ref[...], preferred_element_type=jnp.float32)
    o_ref[...] = h * dis_ref[...]


def _layer2_body(lo_ref, hi_ref, dis_ref, b_ref, wt_ref, o_ref):
    acc = jnp.concatenate([lo_ref[...], hi_ref[...]], axis=1)
    z = jnp.maximum(acc * dis_ref[...] + b_ref[...], 0.0)
    h = jnp.dot(z, wt_ref[...], preferred_element_type=jnp.float32)
    o_ref[...] = h * dis_ref[...]


def _final_body(lo_ref, hi_ref, dis_ref, b_ref, wt_ref, bfc_ref, o_ref):
    acc = jnp.concatenate([lo_ref[...], hi_ref[...]], axis=1)
    z = jnp.maximum(acc * dis_ref[...] + b_ref[...], 0.0)
    m = jnp.sum(z, axis=0, keepdims=True) * (1.0 / N)
    o_ref[...] = jnp.dot(m, wt_ref[...],
                         preferred_element_type=jnp.float32) + bfc_ref[...]


_dis_call = pl.pallas_call(
    _dis_body,
    out_shape=jax.ShapeDtypeStruct((DEG_OUT // 128, 128), jnp.float32),
)

_mm_call = pl.pallas_call(
    _mm_scale_body,
    out_shape=jax.ShapeDtypeStruct((N, F), jnp.float32),
)

_layer2_call = pl.pallas_call(
    _layer2_body,
    out_shape=jax.ShapeDtypeStruct((N, F), jnp.float32),
)

_final_call = pl.pallas_call(
    _final_body,
    out_shape=jax.ShapeDtypeStruct((1, F), jnp.float32),
)


def kernel(x, edge_index, W1, b1, W2, b2, Wfc, bfc):
    src = edge_index[0].astype(jnp.int32)
    dst = edge_index[1].astype(jnp.int32)
    src_p = jnp.concatenate([src, jnp.zeros((EPAD - E,), jnp.int32)])
    dst_p = jnp.concatenate([dst, jnp.full((EPAD - E,), TRASH, jnp.int32)])

    d0, d1 = _deg_call(dst_p)
    dis = _dis_call(d0.reshape(DEG_OUT // 128, 128),
                    d1.reshape(DEG_OUT // 128, 128))
    dis_col = dis.reshape(DEG_OUT)[:N].reshape(N, 1)

    h1 = _mm_call(x, W1.T, dis_col)
    a1 = _scatter_call(h1.reshape(2 * N, H), src_p, dst_p)
    h2 = _layer2_call(a1[0], a1[1], dis_col, b1.reshape(1, F), W2.T)
    a2 = _scatter_call(h2.reshape(2 * N, H), src_p, dst_p)
    out = _final_call(a2[0], a2[1], dis_col, b2.reshape(1, F),
                      Wfc.T, bfc.reshape(1, F))
    return out


# speedup vs baseline: 10.7847x; 1.1434x over previous
import jax
import jax.numpy as jnp
from jax import lax
from jax.experimental import pallas as pl
from jax.experimental.pallas import tpu as pltpu
from jax.experimental.pallas import tpu_sc as plsc

N = 10000
E = 320000
F = 128
H = 64
NC, NS, L = 2, 16, 16
CH = 128
EPAD = 327680
NCH = EPAD // NS // CH
NBUF = 4
TRASH = N
ACC_ROWS = 10240
DEG_W = 80896
DEG_OUT = 10112

_mesh = plsc.VectorSubcoreMesh(
    core_axis_name="c", subcore_axis_name="s", num_cores=NC, num_subcores=NS)


def _deg_body(dst_hbm, out0_hbm, out1_hbm, deg_sh, zb_v, ones_v, dst_v):
    c = lax.axis_index("c")
    s = lax.axis_index("s")

    def _zb(k, carry):
        zb_v[pl.ds(k * L, L)] = jnp.zeros((L,), jnp.float32)
        return carry
    lax.fori_loop(0, DEG_OUT // L, _zb, 0)

    def _ob(k, carry):
        ones_v[pl.ds(k * L, L)] = jnp.ones((L,), jnp.float32)
        return carry
    lax.fori_loop(0, CH // L, _ob, 0)
    pltpu.sync_copy(zb_v.at[pl.ds(0, DEG_W // NS)],
                    deg_sh.at[pl.ds(s * (DEG_W // NS), DEG_W // NS)])
    plsc.subcore_barrier()

    e0 = c * (EPAD // NC) + s * (EPAD // (NC * NS))

    def _chunk(i, carry):
        pltpu.sync_copy(dst_hbm.at[pl.ds(e0 + i * CH, CH)], dst_v)
        pltpu.sync_copy(ones_v, deg_sh.at[dst_v], add=True)
        return carry
    lax.fori_loop(0, EPAD // (NC * NS) // CH, _chunk, 0)
    plsc.subcore_barrier()

    @pl.when(s == 0)
    def _():
        pltpu.sync_copy(deg_sh.at[pl.ds(0, DEG_OUT)], zb_v)

        @pl.when(c == 0)
        def _():
            pltpu.sync_copy(zb_v, out0_hbm)

        @pl.when(c == 1)
        def _():
            pltpu.sync_copy(zb_v, out1_hbm)


_deg_call = pl.kernel(
    _deg_body,
    out_type=(jax.ShapeDtypeStruct((DEG_OUT,), jnp.float32),
              jax.ShapeDtypeStruct((DEG_OUT,), jnp.float32)),
    mesh=_mesh,
    scratch_types=[
        pltpu.VMEM_SHARED((DEG_W,), jnp.float32),
        pltpu.VMEM((DEG_OUT,), jnp.float32),
        pltpu.VMEM((CH,), jnp.float32),
        pltpu.VMEM((CH,), jnp.int32),
    ],
)


def _scatter_body(h2_hbm, src_hbm, dst_hbm, out_hbm,
                  acc_sh, sidx_v, didx_v, rows0_v, rows1_v, rows2_v, rows3_v,
                  g0, g1, g2, g3, s0, s1, s2, s3):
    c = lax.axis_index("c")
    s = lax.axis_index("s")
    rowbufs = (rows0_v, rows1_v, rows2_v, rows3_v)
    gsems = (g0, g1, g2, g3)
    ssems = (s0, s1, s2, s3)

    r0 = s * (ACC_ROWS // NS)

    def _init(i, carry):
        base = r0 + i * CH
        for j in range(CH // L):
            v = base + j * L + lax.iota(jnp.int32, L)
            v = jnp.minimum(v, N - 1)
            sidx_v[0, pl.ds(j * L, L)] = v * 2 + c
        pltpu.sync_copy(h2_hbm.at[sidx_v.at[0]], rows0_v)
        pltpu.sync_copy(rows0_v, acc_sh.at[pl.ds(base, CH)])
        return carry
    lax.fori_loop(0, (ACC_ROWS // NS) // CH, _init, 0)

    pltpu.sync_copy(src_hbm.at[pl.ds(s * NCH, NCH)], sidx_v)
    pltpu.sync_copy(dst_hbm.at[pl.ds(s * NCH, NCH)], didx_v)

    def _tr(i, carry):
        def _trj(j, carry2):
            v = sidx_v[i, pl.ds(j * L, L)]
            sidx_v[i, pl.ds(j * L, L)] = v * 2 + c
            return carry2
        return lax.fori_loop(0, CH // L, _trj, carry)
    lax.fori_loop(0, NCH, _tr, 0)
    plsc.subcore_barrier()

    for b in range(NBUF):
        pltpu.async_copy(h2_hbm.at[sidx_v.at[b]], rowbufs[b], gsems[b])

    def _ring(g, carry):
        for b in range(NBUF):
            i = g * NBUF + b
            pltpu.make_async_copy(
                h2_hbm.at[sidx_v.at[i]], rowbufs[b], gsems[b]).wait()
            pltpu.async_copy(
                rowbufs[b], acc_sh.at[didx_v.at[i]], ssems[b], add=True)
            pltpu.make_async_copy(
                rowbufs[b], acc_sh.at[didx_v.at[i]], ssems[b]).wait()

            @pl.when(i + NBUF < NCH)
            def _():
                pltpu.async_copy(
                    h2_hbm.at[sidx_v.at[i + NBUF]], rowbufs[b], gsems[b])
        return carry
    lax.fori_loop(0, NCH // NBUF, _ring, 0)
    plsc.subcore_barrier()

    QF = 632
    QL = N - 15 * QF
    w0 = s * QF

    @pl.when(s < 15)
    def _():
        pltpu.sync_copy(acc_sh.at[pl.ds(w0, QF)], out_hbm.at[c, pl.ds(w0, QF)])

    @pl.when(s == 15)
    def _():
        pltpu.sync_copy(acc_sh.at[pl.ds(w0, QL)], out_hbm.at[c, pl.ds(w0, QL)])


_scatter_call = pl.kernel(
    _scatter_body,
    out_type=jax.ShapeDtypeStruct((NC, N, H), jnp.float32),
    mesh=_mesh,
    compiler_params=pltpu.CompilerParams(use_tc_tiling_on_sc=False),
    scratch_types=[
        pltpu.VMEM_SHARED((ACC_ROWS, H), jnp.float32),
        pltpu.VMEM((NCH, CH), jnp.int32),
        pltpu.VMEM((NCH, CH), jnp.int32),
        pltpu.VMEM((CH, H), jnp.float32),
        pltpu.VMEM((CH, H), jnp.float32),
        pltpu.VMEM((CH, H), jnp.float32),
        pltpu.VMEM((CH, H), jnp.float32),
        pltpu.SemaphoreType.DMA,
        pltpu.SemaphoreType.DMA,
        pltpu.SemaphoreType.DMA,
        pltpu.SemaphoreType.DMA,
        pltpu.SemaphoreType.DMA,
        pltpu.SemaphoreType.DMA,
        pltpu.SemaphoreType.DMA,
        pltpu.SemaphoreType.DMA,
    ],
)


def _dis_body(d0_ref, d1_ref, dis_ref):
    deg = d0_ref[...] + d1_ref[...] + 1.0
    dis_ref[...] = lax.rsqrt(deg)


def _mm_scale_body(x_ref, wt_ref, dis_ref, o_ref):
    h = jnp.dot(x_ref[...], wt_ref[...], preferred_element_type=jnp.float32)
    o_ref[...] = h * dis_ref[...]


def _layer2_body(lo_ref, hi_ref, dis_ref, b_ref, wt_ref, o_ref):
    acc = jnp.concatenate([lo_ref[...], hi_ref[...]], axis=1)
    z = jnp.maximum(acc * dis_ref[...] + b_ref[...], 0.0)
    h = jnp.dot(z, wt_ref[...], preferred_element_type=jnp.float32)
    o_ref[...] = h * dis_ref[...]


def _final_body(lo_ref, hi_ref, dis_ref, b_ref, wt_ref, bfc_ref, o_ref):
    acc = jnp.concatenate([lo_ref[...], hi_ref[...]], axis=1)
    z = jnp.maximum(acc * dis_ref[...] + b_ref[...], 0.0)
    m = jnp.sum(z, axis=0, keepdims=True) * (1.0 / N)
    o_ref[...] = jnp.dot(m, wt_ref[...],
                         preferred_element_type=jnp.float32) + bfc_ref[...]


_dis_call = pl.pallas_call(
    _dis_body,
    out_shape=jax.ShapeDtypeStruct((DEG_OUT // 128, 128), jnp.float32),
)

_mm_call = pl.pallas_call(
    _mm_scale_body,
    out_shape=jax.ShapeDtypeStruct((N, F), jnp.float32),
)

_layer2_call = pl.pallas_call(
    _layer2_body,
    out_shape=jax.ShapeDtypeStruct((N, F), jnp.float32),
)

_final_call = pl.pallas_call(
    _final_body,
    out_shape=jax.ShapeDtypeStruct((1, F), jnp.float32),
)


def kernel(x, edge_index, W1, b1, W2, b2, Wfc, bfc):
    src = edge_index[0].astype(jnp.int32)
    dst = edge_index[1].astype(jnp.int32)
    src_p = jnp.concatenate([src, jnp.zeros((EPAD - E,), jnp.int32)])
    dst_p = jnp.concatenate([dst, jnp.full((EPAD - E,), TRASH, jnp.int32)])

    d0, d1 = _deg_call(dst_p)
    dis = _dis_call(d0.reshape(DEG_OUT // 128, 128),
                    d1.reshape(DEG_OUT // 128, 128))
    dis_col = dis.reshape(DEG_OUT)[:N].reshape(N, 1)

    src2 = src_p.reshape(EPAD // CH, CH)
    dst2 = dst_p.reshape(EPAD // CH, CH)
    h1 = _mm_call(x, W1.T, dis_col)
    a1 = _scatter_call(h1.reshape(2 * N, H), src2, dst2)
    h2 = _layer2_call(a1[0], a1[1], dis_col, b1.reshape(1, F), W2.T)
    a2 = _scatter_call(h2.reshape(2 * N, H), src2, dst2)
    out = _final_call(a2[0], a2[1], dis_col, b2.reshape(1, F),
                      Wfc.T, bfc.reshape(1, F))
    return out


# speedup vs baseline: 20.7445x; 1.9235x over previous
import jax
import jax.numpy as jnp
from jax import lax
from jax.experimental import pallas as pl
from jax.experimental.pallas import tpu as pltpu
from jax.experimental.pallas import tpu_sc as plsc

N = 10000
E = 320000
F = 128
H = 64
NC, NS, L = 2, 16, 16
CH = 128
EPAD = 327680
NCH = EPAD // NS // CH
NBUF = 4
TRASH = N
ACC_ROWS = 10240
DEG_W = 80896
DEG_OUT = 10112

_mesh = plsc.VectorSubcoreMesh(
    core_axis_name="c", subcore_axis_name="s", num_cores=NC, num_subcores=NS)


def _deg_body(dst_hbm, out0_hbm, out1_hbm, deg_sh, zb_v, ones_v, dst_v):
    c = lax.axis_index("c")
    s = lax.axis_index("s")

    def _zb(k, carry):
        zb_v[pl.ds(k * L, L)] = jnp.zeros((L,), jnp.float32)
        return carry
    lax.fori_loop(0, DEG_OUT // L, _zb, 0)

    def _ob(k, carry):
        ones_v[pl.ds(k * L, L)] = jnp.ones((L,), jnp.float32)
        return carry
    lax.fori_loop(0, CH // L, _ob, 0)
    pltpu.sync_copy(zb_v.at[pl.ds(0, DEG_W // NS)],
                    deg_sh.at[pl.ds(s * (DEG_W // NS), DEG_W // NS)])
    plsc.subcore_barrier()

    e0 = c * (EPAD // NC) + s * (EPAD // (NC * NS))

    def _chunk(i, carry):
        pltpu.sync_copy(dst_hbm.at[pl.ds(e0 + i * CH, CH)], dst_v)
        pltpu.sync_copy(ones_v, deg_sh.at[dst_v], add=True)
        return carry
    lax.fori_loop(0, EPAD // (NC * NS) // CH, _chunk, 0)
    plsc.subcore_barrier()

    @pl.when(s == 0)
    def _():
        pltpu.sync_copy(deg_sh.at[pl.ds(0, DEG_OUT)], zb_v)

        @pl.when(c == 0)
        def _():
            pltpu.sync_copy(zb_v, out0_hbm)

        @pl.when(c == 1)
        def _():
            pltpu.sync_copy(zb_v, out1_hbm)


_deg_call = pl.kernel(
    _deg_body,
    out_type=(jax.ShapeDtypeStruct((DEG_OUT,), jnp.float32),
              jax.ShapeDtypeStruct((DEG_OUT,), jnp.float32)),
    mesh=_mesh,
    scratch_types=[
        pltpu.VMEM_SHARED((DEG_W,), jnp.float32),
        pltpu.VMEM((DEG_OUT,), jnp.float32),
        pltpu.VMEM((CH,), jnp.float32),
        pltpu.VMEM((CH,), jnp.int32),
    ],
)


def _scatter_body(h2_hbm, src_hbm, dst_hbm, out_hbm,
                  acc_sh, tab_sh, sidx_v, didx_v,
                  rows0_v, rows1_v, rows2_v, rows3_v,
                  g0, g1, g2, g3, s0, s1, s2, s3):
    c = lax.axis_index("c")
    s = lax.axis_index("s")
    rowbufs = (rows0_v, rows1_v, rows2_v, rows3_v)
    gsems = (g0, g1, g2, g3)
    ssems = (s0, s1, s2, s3)

    r0 = s * (ACC_ROWS // NS)

    def _init(i, carry):
        base = r0 + i * CH
        for j in range(CH // L):
            v = base + j * L + lax.iota(jnp.int32, L)
            v = jnp.minimum(v, N - 1)
            sidx_v[0, pl.ds(j * L, L)] = v * 2 + c
        pltpu.sync_copy(h2_hbm.at[sidx_v.at[0]], rows0_v)
        pltpu.sync_copy(rows0_v, acc_sh.at[pl.ds(base, CH)])
        pltpu.sync_copy(rows0_v, tab_sh.at[pl.ds(base, CH)])
        return carry
    lax.fori_loop(0, (ACC_ROWS // NS) // CH, _init, 0)
    plsc.subcore_barrier()

    NQ = NCH // 4

    def _quarter(q, carry):
        qbase = s * NCH + q * NQ
        pltpu.sync_copy(src_hbm.at[pl.ds(qbase, NQ)], sidx_v)
        pltpu.sync_copy(dst_hbm.at[pl.ds(qbase, NQ)], didx_v)
        for b in range(NBUF):
            pltpu.async_copy(tab_sh.at[sidx_v.at[b]], rowbufs[b], gsems[b])

        def _ring(g, carry2):
            for b in range(NBUF):
                i = g * NBUF + b
                pltpu.make_async_copy(
                    tab_sh.at[sidx_v.at[i]], rowbufs[b], gsems[b]).wait()
                pltpu.async_copy(
                    rowbufs[b], acc_sh.at[didx_v.at[i]], ssems[b], add=True)
                pltpu.make_async_copy(
                    rowbufs[b], acc_sh.at[didx_v.at[i]], ssems[b]).wait()

                @pl.when(i + NBUF < NQ)
                def _():
                    pltpu.async_copy(
                        tab_sh.at[sidx_v.at[i + NBUF]], rowbufs[b], gsems[b])
            return carry2
        return lax.fori_loop(0, NQ // NBUF, _ring, carry)
    lax.fori_loop(0, 4, _quarter, 0)
    plsc.subcore_barrier()

    QF = 632
    QL = N - 15 * QF
    w0 = s * QF

    @pl.when(s < 15)
    def _():
        pltpu.sync_copy(acc_sh.at[pl.ds(w0, QF)], out_hbm.at[c, pl.ds(w0, QF)])

    @pl.when(s == 15)
    def _():
        pltpu.sync_copy(acc_sh.at[pl.ds(w0, QL)], out_hbm.at[c, pl.ds(w0, QL)])


_scatter_call = pl.kernel(
    _scatter_body,
    out_type=jax.ShapeDtypeStruct((NC, N, H), jnp.float32),
    mesh=_mesh,
    compiler_params=pltpu.CompilerParams(use_tc_tiling_on_sc=False),
    scratch_types=[
        pltpu.VMEM_SHARED((ACC_ROWS, H), jnp.float32),
        pltpu.VMEM_SHARED((ACC_ROWS, H), jnp.float32),
        pltpu.VMEM((NCH // 4, CH), jnp.int32),
        pltpu.VMEM((NCH // 4, CH), jnp.int32),
        pltpu.VMEM((CH, H), jnp.float32),
        pltpu.VMEM((CH, H), jnp.float32),
        pltpu.VMEM((CH, H), jnp.float32),
        pltpu.VMEM((CH, H), jnp.float32),
        pltpu.SemaphoreType.DMA,
        pltpu.SemaphoreType.DMA,
        pltpu.SemaphoreType.DMA,
        pltpu.SemaphoreType.DMA,
        pltpu.SemaphoreType.DMA,
        pltpu.SemaphoreType.DMA,
        pltpu.SemaphoreType.DMA,
        pltpu.SemaphoreType.DMA,
    ],
)


def _dis_body(d0_ref, d1_ref, dis_ref):
    deg = d0_ref[...] + d1_ref[...] + 1.0
    dis_ref[...] = lax.rsqrt(deg)


def _mm_scale_body(x_ref, wt_ref, dis_ref, o_ref):
    h = jnp.dot(x_ref[...], wt_ref[...], preferred_element_type=jnp.float32)
    o_ref[...] = h * dis_ref[...]


def _layer2_body(lo_ref, hi_ref, dis_ref, b_ref, wt_ref, o_ref):
    acc = jnp.concatenate([lo_ref[...], hi_ref[...]], axis=1)
    z = jnp.maximum(acc * dis_ref[...] + b_ref[...], 0.0)
    h = jnp.dot(z, wt_ref[...], preferred_element_type=jnp.float32)
    o_ref[...] = h * dis_ref[...]


def _final_body(lo_ref, hi_ref, dis_ref, b_ref, wt_ref, bfc_ref, o_ref):
    acc = jnp.concatenate([lo_ref[...], hi_ref[...]], axis=1)
    z = jnp.maximum(acc * dis_ref[...] + b_ref[...], 0.0)
    m = jnp.sum(z, axis=0, keepdims=True) * (1.0 / N)
    o_ref[...] = jnp.dot(m, wt_ref[...],
                         preferred_element_type=jnp.float32) + bfc_ref[...]


_dis_call = pl.pallas_call(
    _dis_body,
    out_shape=jax.ShapeDtypeStruct((DEG_OUT // 128, 128), jnp.float32),
)

_mm_call = pl.pallas_call(
    _mm_scale_body,
    out_shape=jax.ShapeDtypeStruct((N, F), jnp.float32),
)

_layer2_call = pl.pallas_call(
    _layer2_body,
    out_shape=jax.ShapeDtypeStruct((N, F), jnp.float32),
)

_final_call = pl.pallas_call(
    _final_body,
    out_shape=jax.ShapeDtypeStruct((1, F), jnp.float32),
)


def kernel(x, edge_index, W1, b1, W2, b2, Wfc, bfc):
    src = edge_index[0].astype(jnp.int32)
    dst = edge_index[1].astype(jnp.int32)
    src_p = jnp.concatenate([src, jnp.zeros((EPAD - E,), jnp.int32)])
    dst_p = jnp.concatenate([dst, jnp.full((EPAD - E,), TRASH, jnp.int32)])

    d0, d1 = _deg_call(dst_p)
    dis = _dis_call(d0.reshape(DEG_OUT // 128, 128),
                    d1.reshape(DEG_OUT // 128, 128))
    dis_col = dis.reshape(DEG_OUT)[:N].reshape(N, 1)

    src2 = src_p.reshape(EPAD // CH, CH)
    dst2 = dst_p.reshape(EPAD // CH, CH)
    h1 = _mm_call(x, W1.T, dis_col)
    a1 = _scatter_call(h1.reshape(2 * N, H), src2, dst2)
    h2 = _layer2_call(a1[0], a1[1], dis_col, b1.reshape(1, F), W2.T)
    a2 = _scatter_call(h2.reshape(2 * N, H), src2, dst2)
    out = _final_call(a2[0], a2[1], dis_col, b2.reshape(1, F),
                      Wfc.T, bfc.reshape(1, F))
    return out


# speedup vs baseline: 23.9924x; 1.1566x over previous
import jax
import jax.numpy as jnp
from jax import lax
from jax.experimental import pallas as pl
from jax.experimental.pallas import tpu as pltpu
from jax.experimental.pallas import tpu_sc as plsc

N = 10000
E = 320000
F = 128
H = 64
NC, NS, L = 2, 16, 16
CH = 128
EPAD = 327680
NCH = EPAD // NS // CH
NBUF = 4
TRASH = N
ACC_ROWS = 10240
DEG_W = 80896
DEG_OUT = 10112

_mesh = plsc.VectorSubcoreMesh(
    core_axis_name="c", subcore_axis_name="s", num_cores=NC, num_subcores=NS)


def _deg_body(dst_hbm, out0_hbm, out1_hbm, deg_sh, zb_v, ones_v, dst_v):
    c = lax.axis_index("c")
    s = lax.axis_index("s")

    def _zb(k, carry):
        zb_v[pl.ds(k * L, L)] = jnp.zeros((L,), jnp.float32)
        return carry
    lax.fori_loop(0, DEG_OUT // L, _zb, 0)

    def _ob(k, carry):
        ones_v[pl.ds(k * L, L)] = jnp.ones((L,), jnp.float32)
        return carry
    lax.fori_loop(0, CH // L, _ob, 0)
    pltpu.sync_copy(zb_v.at[pl.ds(0, DEG_W // NS)],
                    deg_sh.at[pl.ds(s * (DEG_W // NS), DEG_W // NS)])
    plsc.subcore_barrier()

    e0 = c * (EPAD // NC) + s * (EPAD // (NC * NS))

    def _chunk(i, carry):
        pltpu.sync_copy(dst_hbm.at[pl.ds(e0 + i * CH, CH)], dst_v)
        pltpu.sync_copy(ones_v, deg_sh.at[dst_v], add=True)
        return carry
    lax.fori_loop(0, EPAD // (NC * NS) // CH, _chunk, 0)
    plsc.subcore_barrier()

    @pl.when(s == 0)
    def _():
        pltpu.sync_copy(deg_sh.at[pl.ds(0, DEG_OUT)], zb_v)

        @pl.when(c == 0)
        def _():
            pltpu.sync_copy(zb_v, out0_hbm)

        @pl.when(c == 1)
        def _():
            pltpu.sync_copy(zb_v, out1_hbm)


_deg_call = pl.kernel(
    _deg_body,
    out_type=(jax.ShapeDtypeStruct((DEG_OUT,), jnp.float32),
              jax.ShapeDtypeStruct((DEG_OUT,), jnp.float32)),
    mesh=_mesh,
    scratch_types=[
        pltpu.VMEM_SHARED((DEG_W,), jnp.float32),
        pltpu.VMEM((DEG_OUT,), jnp.float32),
        pltpu.VMEM((CH,), jnp.float32),
        pltpu.VMEM((CH,), jnp.int32),
    ],
)


def _scatter_body(h2_hbm, src_hbm, dst_hbm, out_hbm,
                  acc_sh, tab_sh, sidx_v, didx_v,
                  rows0_v, rows1_v, rows2_v, rows3_v,
                  g0, g1, g2, g3, s0, s1, s2, s3):
    c = lax.axis_index("c")
    s = lax.axis_index("s")
    rowbufs = (rows0_v, rows1_v, rows2_v, rows3_v)
    gsems = (g0, g1, g2, g3)
    ssems = (s0, s1, s2, s3)

    r0 = s * (ACC_ROWS // NS)

    def _init(i, carry):
        base = r0 + i * CH
        for j in range(CH // L):
            v = base + j * L + lax.iota(jnp.int32, L)
            v = jnp.minimum(v, N - 1)
            sidx_v[0, pl.ds(j * L, L)] = v * 2 + c
        pltpu.sync_copy(h2_hbm.at[sidx_v.at[0]], rows0_v)
        pltpu.sync_copy(rows0_v, acc_sh.at[pl.ds(base, CH)])
        pltpu.sync_copy(rows0_v, tab_sh.at[pl.ds(base, CH)])
        return carry
    lax.fori_loop(0, (ACC_ROWS // NS) // CH, _init, 0)
    plsc.subcore_barrier()

    NQ = NCH // 4

    D = 2

    def _quarter(q, carry):
        qbase = s * NCH + q * NQ
        pltpu.sync_copy(src_hbm.at[pl.ds(qbase, NQ)], sidx_v)
        pltpu.sync_copy(dst_hbm.at[pl.ds(qbase, NQ)], didx_v)

        def _slots(g, carry2):
            for b in range(NBUF):
                t = g * NBUF + b
                bs = (b + NBUF - D) % NBUF

                @pl.when(t >= NBUF)
                def _():
                    pltpu.make_async_copy(
                        rowbufs[b], acc_sh.at[didx_v.at[t - NBUF]],
                        ssems[b]).wait()

                @pl.when(t < NQ)
                def _():
                    pltpu.async_copy(
                        tab_sh.at[sidx_v.at[t]], rowbufs[b], gsems[b])

                @pl.when(jnp.logical_and(t >= D, t < NQ + D))
                def _():
                    pltpu.make_async_copy(
                        tab_sh.at[sidx_v.at[t - D]], rowbufs[bs],
                        gsems[bs]).wait()
                    pltpu.async_copy(
                        rowbufs[bs], acc_sh.at[didx_v.at[t - D]],
                        ssems[bs], add=True)
            return carry2
        return lax.fori_loop(0, (NQ + NBUF) // NBUF, _slots, carry)
    lax.fori_loop(0, 4, _quarter, 0)
    plsc.subcore_barrier()

    QF = 632
    QL = N - 15 * QF
    w0 = s * QF

    @pl.when(s < 15)
    def _():
        pltpu.sync_copy(acc_sh.at[pl.ds(w0, QF)], out_hbm.at[c, pl.ds(w0, QF)])

    @pl.when(s == 15)
    def _():
        pltpu.sync_copy(acc_sh.at[pl.ds(w0, QL)], out_hbm.at[c, pl.ds(w0, QL)])


_scatter_call = pl.kernel(
    _scatter_body,
    out_type=jax.ShapeDtypeStruct((NC, N, H), jnp.float32),
    mesh=_mesh,
    compiler_params=pltpu.CompilerParams(use_tc_tiling_on_sc=False),
    scratch_types=[
        pltpu.VMEM_SHARED((ACC_ROWS, H), jnp.float32),
        pltpu.VMEM_SHARED((ACC_ROWS, H), jnp.float32),
        pltpu.VMEM((NCH // 4, CH), jnp.int32),
        pltpu.VMEM((NCH // 4, CH), jnp.int32),
        pltpu.VMEM((CH, H), jnp.float32),
        pltpu.VMEM((CH, H), jnp.float32),
        pltpu.VMEM((CH, H), jnp.float32),
        pltpu.VMEM((CH, H), jnp.float32),
        pltpu.SemaphoreType.DMA,
        pltpu.SemaphoreType.DMA,
        pltpu.SemaphoreType.DMA,
        pltpu.SemaphoreType.DMA,
        pltpu.SemaphoreType.DMA,
        pltpu.SemaphoreType.DMA,
        pltpu.SemaphoreType.DMA,
        pltpu.SemaphoreType.DMA,
    ],
)


def _dis_body(d0_ref, d1_ref, dis_ref):
    deg = d0_ref[...] + d1_ref[...] + 1.0
    dis_ref[...] = lax.rsqrt(deg)


def _mm_scale_body(x_ref, wt_ref, dis_ref, o_ref):
    h = jnp.dot(x_ref[...], wt_ref[...], preferred_element_type=jnp.float32)
    o_ref[...] = h * dis_ref[...]


def _layer2_body(lo_ref, hi_ref, dis_ref, b_ref, wt_ref, o_ref):
    acc = jnp.concatenate([lo_ref[...], hi_ref[...]], axis=1)
    z = jnp.maximum(acc * dis_ref[...] + b_ref[...], 0.0)
    h = jnp.dot(z, wt_ref[...], preferred_element_type=jnp.float32)
    o_ref[...] = h * dis_ref[...]


def _final_body(lo_ref, hi_ref, dis_ref, b_ref, wt_ref, bfc_ref, o_ref):
    acc = jnp.concatenate([lo_ref[...], hi_ref[...]], axis=1)
    z = jnp.maximum(acc * dis_ref[...] + b_ref[...], 0.0)
    m = jnp.sum(z, axis=0, keepdims=True) * (1.0 / N)
    o_ref[...] = jnp.dot(m, wt_ref[...],
                         preferred_element_type=jnp.float32) + bfc_ref[...]


_dis_call = pl.pallas_call(
    _dis_body,
    out_shape=jax.ShapeDtypeStruct((DEG_OUT // 128, 128), jnp.float32),
)

_mm_call = pl.pallas_call(
    _mm_scale_body,
    out_shape=jax.ShapeDtypeStruct((N, F), jnp.float32),
)

_layer2_call = pl.pallas_call(
    _layer2_body,
    out_shape=jax.ShapeDtypeStruct((N, F), jnp.float32),
)

_final_call = pl.pallas_call(
    _final_body,
    out_shape=jax.ShapeDtypeStruct((1, F), jnp.float32),
)


def kernel(x, edge_index, W1, b1, W2, b2, Wfc, bfc):
    src = edge_index[0].astype(jnp.int32)
    dst = edge_index[1].astype(jnp.int32)
    src_p = jnp.concatenate([src, jnp.zeros((EPAD - E,), jnp.int32)])
    dst_p = jnp.concatenate([dst, jnp.full((EPAD - E,), TRASH, jnp.int32)])

    d0, d1 = _deg_call(dst_p)
    dis = _dis_call(d0.reshape(DEG_OUT // 128, 128),
                    d1.reshape(DEG_OUT // 128, 128))
    dis_col = dis.reshape(DEG_OUT)[:N].reshape(N, 1)

    src2 = src_p.reshape(EPAD // CH, CH)
    dst2 = dst_p.reshape(EPAD // CH, CH)
    h1 = _mm_call(x, W1.T, dis_col)
    a1 = _scatter_call(h1.reshape(2 * N, H), src2, dst2)
    h2 = _layer2_call(a1[0], a1[1], dis_col, b1.reshape(1, F), W2.T)
    a2 = _scatter_call(h2.reshape(2 * N, H), src2, dst2)
    out = _final_call(a2[0], a2[1], dis_col, b2.reshape(1, F),
                      Wfc.T, bfc.reshape(1, F))
    return out


# speedup vs baseline: 26.3047x; 1.0964x over previous
import jax
import jax.numpy as jnp
from jax import lax
from jax.experimental import pallas as pl
from jax.experimental.pallas import tpu as pltpu
from jax.experimental.pallas import tpu_sc as plsc

N = 10000
E = 320000
F = 128
H = 64
NC, NS, L = 2, 16, 16
CH = 128
EPAD = 327680
NCH = EPAD // NS // CH
NBUF = 4
TRASH = N
ACC_ROWS = 10240
DEG_W = 80896
DEG_OUT = 10112

_mesh = plsc.VectorSubcoreMesh(
    core_axis_name="c", subcore_axis_name="s", num_cores=NC, num_subcores=NS)


def _deg_body(dst_hbm, out0_hbm, out1_hbm, deg_sh, zb_v, ones_v, dstb_v, dsem):
    c = lax.axis_index("c")
    s = lax.axis_index("s")
    w = c * NS + s
    ROWS = EPAD // CH // (NC * NS)
    ZQ = DEG_OUT // NS

    def _z(k, carry):
        zb_v[pl.ds(k * L, L)] = jnp.zeros((L,), jnp.float32)
        return carry
    lax.fori_loop(0, 640 // L, _z, 0)

    def _ob(k, carry):
        ones_v[pl.ds(k * L, L)] = jnp.ones((L,), jnp.float32)
        return carry
    lax.fori_loop(0, CH // L, _ob, 0)
    pltpu.sync_copy(zb_v.at[pl.ds(0, ZQ)], deg_sh.at[pl.ds(s * ZQ, ZQ)])
    pltpu.sync_copy(dst_hbm.at[pl.ds(w * ROWS, ROWS)], dstb_v)
    plsc.subcore_barrier()

    def _grp(g, carry):
        for b in range(8):
            pltpu.async_copy(
                ones_v, deg_sh.at[dstb_v.at[g * 8 + b]], dsem, add=True)
        for b in range(8):
            pltpu.make_async_copy(
                ones_v, deg_sh.at[dstb_v.at[g * 8 + b]], dsem).wait()
        return carry
    lax.fori_loop(0, ROWS // 8, _grp, 0)
    plsc.subcore_barrier()

    @pl.when(s == 0)
    def _():
        @pl.when(c == 0)
        def _():
            pltpu.sync_copy(deg_sh, out0_hbm)

        @pl.when(c == 1)
        def _():
            pltpu.sync_copy(deg_sh, out1_hbm)


_deg_call = pl.kernel(
    _deg_body,
    out_type=(jax.ShapeDtypeStruct((DEG_OUT,), jnp.float32),
              jax.ShapeDtypeStruct((DEG_OUT,), jnp.float32)),
    mesh=_mesh,
    compiler_params=pltpu.CompilerParams(use_tc_tiling_on_sc=False),
    scratch_types=[
        pltpu.VMEM_SHARED((DEG_OUT,), jnp.float32),
        pltpu.VMEM((640,), jnp.float32),
        pltpu.VMEM((CH,), jnp.float32),
        pltpu.VMEM((EPAD // CH // (NC * NS), CH), jnp.int32),
        pltpu.SemaphoreType.DMA,
    ],
)


def _scatter_body(h2_hbm, src_hbm, dst_hbm, out_hbm,
                  acc_sh, tab_sh, sidx_v, didx_v,
                  rows0_v, rows1_v, rows2_v, rows3_v,
                  g0, g1, g2, g3, s0, s1, s2, s3):
    c = lax.axis_index("c")
    s = lax.axis_index("s")
    rowbufs = (rows0_v, rows1_v, rows2_v, rows3_v)
    gsems = (g0, g1, g2, g3)
    ssems = (s0, s1, s2, s3)

    r0 = s * (ACC_ROWS // NS)

    def _init(i, carry):
        base = r0 + i * CH
        for j in range(CH // L):
            v = base + j * L + lax.iota(jnp.int32, L)
            v = jnp.minimum(v, N - 1)
            sidx_v[0, pl.ds(j * L, L)] = v * 2 + c
        pltpu.sync_copy(h2_hbm.at[sidx_v.at[0]], rows0_v)
        pltpu.sync_copy(rows0_v, acc_sh.at[pl.ds(base, CH)])
        pltpu.sync_copy(rows0_v, tab_sh.at[pl.ds(base, CH)])
        return carry
    lax.fori_loop(0, (ACC_ROWS // NS) // CH, _init, 0)
    plsc.subcore_barrier()

    NQ = NCH // 4

    D = 2

    def _quarter(q, carry):
        qbase = s * NCH + q * NQ
        pltpu.sync_copy(src_hbm.at[pl.ds(qbase, NQ)], sidx_v)
        pltpu.sync_copy(dst_hbm.at[pl.ds(qbase, NQ)], didx_v)

        def _slots(g, carry2):
            for b in range(NBUF):
                t = g * NBUF + b
                bs = (b + NBUF - D) % NBUF

                @pl.when(t >= NBUF)
                def _():
                    pltpu.make_async_copy(
                        rowbufs[b], acc_sh.at[didx_v.at[t - NBUF]],
                        ssems[b]).wait()

                @pl.when(t < NQ)
                def _():
                    pltpu.async_copy(
                        tab_sh.at[sidx_v.at[t]], rowbufs[b], gsems[b])

                @pl.when(jnp.logical_and(t >= D, t < NQ + D))
                def _():
                    pltpu.make_async_copy(
                        tab_sh.at[sidx_v.at[t - D]], rowbufs[bs],
                        gsems[bs]).wait()
                    pltpu.async_copy(
                        rowbufs[bs], acc_sh.at[didx_v.at[t - D]],
                        ssems[bs], add=True)
            return carry2
        return lax.fori_loop(0, (NQ + NBUF) // NBUF, _slots, carry)
    lax.fori_loop(0, 4, _quarter, 0)
    plsc.subcore_barrier()

    QF = 632
    QL = N - 15 * QF
    w0 = s * QF

    @pl.when(s < 15)
    def _():
        pltpu.sync_copy(acc_sh.at[pl.ds(w0, QF)], out_hbm.at[c, pl.ds(w0, QF)])

    @pl.when(s == 15)
    def _():
        pltpu.sync_copy(acc_sh.at[pl.ds(w0, QL)], out_hbm.at[c, pl.ds(w0, QL)])


_scatter_call = pl.kernel(
    _scatter_body,
    out_type=jax.ShapeDtypeStruct((NC, N, H), jnp.float32),
    mesh=_mesh,
    compiler_params=pltpu.CompilerParams(use_tc_tiling_on_sc=False),
    scratch_types=[
        pltpu.VMEM_SHARED((ACC_ROWS, H), jnp.float32),
        pltpu.VMEM_SHARED((ACC_ROWS, H), jnp.float32),
        pltpu.VMEM((NCH // 4, CH), jnp.int32),
        pltpu.VMEM((NCH // 4, CH), jnp.int32),
        pltpu.VMEM((CH, H), jnp.float32),
        pltpu.VMEM((CH, H), jnp.float32),
        pltpu.VMEM((CH, H), jnp.float32),
        pltpu.VMEM((CH, H), jnp.float32),
        pltpu.SemaphoreType.DMA,
        pltpu.SemaphoreType.DMA,
        pltpu.SemaphoreType.DMA,
        pltpu.SemaphoreType.DMA,
        pltpu.SemaphoreType.DMA,
        pltpu.SemaphoreType.DMA,
        pltpu.SemaphoreType.DMA,
        pltpu.SemaphoreType.DMA,
    ],
)


def _dis_body(d0_ref, d1_ref, dis_ref):
    deg = d0_ref[...] + d1_ref[...] + 1.0
    dis_ref[...] = lax.rsqrt(deg)


def _mm_scale_body(x_ref, wt_ref, dis_ref, o_ref):
    h = jnp.dot(x_ref[...], wt_ref[...], preferred_element_type=jnp.float32)
    o_ref[...] = h * dis_ref[...]


def _layer2_body(lo_ref, hi_ref, dis_ref, b_ref, wt_ref, o_ref):
    acc = jnp.concatenate([lo_ref[...], hi_ref[...]], axis=1)
    z = jnp.maximum(acc * dis_ref[...] + b_ref[...], 0.0)
    h = jnp.dot(z, wt_ref[...], preferred_element_type=jnp.float32)
    o_ref[...] = h * dis_ref[...]


def _final_body(lo_ref, hi_ref, dis_ref, b_ref, wt_ref, bfc_ref, o_ref):
    acc = jnp.concatenate([lo_ref[...], hi_ref[...]], axis=1)
    z = jnp.maximum(acc * dis_ref[...] + b_ref[...], 0.0)
    m = jnp.sum(z, axis=0, keepdims=True) * (1.0 / N)
    o_ref[...] = jnp.dot(m, wt_ref[...],
                         preferred_element_type=jnp.float32) + bfc_ref[...]


_dis_call = pl.pallas_call(
    _dis_body,
    out_shape=jax.ShapeDtypeStruct((DEG_OUT // 128, 128), jnp.float32),
)

_mm_call = pl.pallas_call(
    _mm_scale_body,
    out_shape=jax.ShapeDtypeStruct((N, F), jnp.float32),
)

_layer2_call = pl.pallas_call(
    _layer2_body,
    out_shape=jax.ShapeDtypeStruct((N, F), jnp.float32),
)

_final_call = pl.pallas_call(
    _final_body,
    out_shape=jax.ShapeDtypeStruct((1, F), jnp.float32),
)


def kernel(x, edge_index, W1, b1, W2, b2, Wfc, bfc):
    src = edge_index[0].astype(jnp.int32)
    dst = edge_index[1].astype(jnp.int32)
    src_p = jnp.concatenate([src, jnp.zeros((EPAD - E,), jnp.int32)])
    dst_p = jnp.concatenate([dst, jnp.full((EPAD - E,), TRASH, jnp.int32)])

    src2 = src_p.reshape(EPAD // CH, CH)
    dst2 = dst_p.reshape(EPAD // CH, CH)
    d0, d1 = _deg_call(dst2)
    dis = _dis_call(d0.reshape(DEG_OUT // 128, 128),
                    d1.reshape(DEG_OUT // 128, 128))
    dis_col = dis.reshape(DEG_OUT)[:N].reshape(N, 1)
    h1 = _mm_call(x, W1.T, dis_col)
    a1 = _scatter_call(h1.reshape(2 * N, H), src2, dst2)
    h2 = _layer2_call(a1[0], a1[1], dis_col, b1.reshape(1, F), W2.T)
    a2 = _scatter_call(h2.reshape(2 * N, H), src2, dst2)
    out = _final_call(a2[0], a2[1], dis_col, b2.reshape(1, F),
                      Wfc.T, bfc.reshape(1, F))
    return out


# speedup vs baseline: 31.5265x; 1.1985x over previous
import jax
import jax.numpy as jnp
from jax import lax
from jax.experimental import pallas as pl
from jax.experimental.pallas import tpu as pltpu
from jax.experimental.pallas import tpu_sc as plsc

N = 10000
E = 320000
F = 128
H = 64
NC, NS, L = 2, 16, 16
CH = 128
EPAD = 327680
NCH = EPAD // NS // CH
NBUF = 4
TRASH = N
ACC_ROWS = 10240
DEG_W = 80896
DEG_OUT = 10112

_mesh = plsc.VectorSubcoreMesh(
    core_axis_name="c", subcore_axis_name="s", num_cores=NC, num_subcores=NS)


def _deg_body(ei_hbm, out0_hbm, out1_hbm, deg_sh, zb_v, ones_v, dstb_v, dsem):
    c = lax.axis_index("c")
    s = lax.axis_index("s")
    w = c * NS + s
    ROWS = EPAD // CH // (NC * NS)
    ZQ = DEG_OUT // NS

    def _z(k, carry):
        zb_v[pl.ds(k * L, L)] = jnp.zeros((L,), jnp.float32)
        return carry
    lax.fori_loop(0, 640 // L, _z, 0)

    def _ob(k, carry):
        ones_v[pl.ds(k * L, L)] = jnp.ones((L,), jnp.float32)
        return carry
    lax.fori_loop(0, CH // L, _ob, 0)
    pltpu.sync_copy(zb_v.at[pl.ds(0, ZQ)], deg_sh.at[pl.ds(s * ZQ, ZQ)])
    pltpu.sync_copy(ei_hbm.at[1, pl.ds(w * ROWS, ROWS)], dstb_v)
    plsc.subcore_barrier()

    def _grp(g, carry):
        for b in range(8):
            pltpu.async_copy(
                ones_v, deg_sh.at[dstb_v.at[g * 8 + b]], dsem, add=True)
        for b in range(8):
            pltpu.make_async_copy(
                ones_v, deg_sh.at[dstb_v.at[g * 8 + b]], dsem).wait()
        return carry
    lax.fori_loop(0, ROWS // 8, _grp, 0)
    plsc.subcore_barrier()

    @pl.when(s == 0)
    def _():
        @pl.when(c == 0)
        def _():
            pltpu.sync_copy(deg_sh, out0_hbm)

        @pl.when(c == 1)
        def _():
            pltpu.sync_copy(deg_sh, out1_hbm)


_deg_call = pl.kernel(
    _deg_body,
    out_type=(jax.ShapeDtypeStruct((DEG_OUT,), jnp.float32),
              jax.ShapeDtypeStruct((DEG_OUT,), jnp.float32)),
    mesh=_mesh,
    compiler_params=pltpu.CompilerParams(use_tc_tiling_on_sc=False),
    scratch_types=[
        pltpu.VMEM_SHARED((DEG_OUT,), jnp.float32),
        pltpu.VMEM((640,), jnp.float32),
        pltpu.VMEM((CH,), jnp.float32),
        pltpu.VMEM((EPAD // CH // (NC * NS), CH), jnp.int32),
        pltpu.SemaphoreType.DMA,
    ],
)


def _scatter_body(h_hbm, ei_hbm, out_hbm,
                  acc_sh, tab_sh, sidx_v, didx_v,
                  rows0_v, rows1_v, rows2_v, rows3_v,
                  g0, g1, g2, g3, s0, s1, s2, s3):
    c = lax.axis_index("c")
    s = lax.axis_index("s")
    rowbufs = (rows0_v, rows1_v, rows2_v, rows3_v)
    gsems = (g0, g1, g2, g3)
    ssems = (s0, s1, s2, s3)

    r0 = s * (N // NS)
    pltpu.sync_copy(h_hbm.at[pl.ds(r0, N // NS), pl.ds(c * H, H)],
                    tab_sh.at[pl.ds(r0, N // NS)])
    pltpu.sync_copy(h_hbm.at[pl.ds(r0, N // NS), pl.ds(c * H, H)],
                    acc_sh.at[pl.ds(r0, N // NS)])
    plsc.subcore_barrier()

    NQ = NCH // 4

    D = 2

    def _quarter(q, carry):
        qbase = s * NCH + q * NQ
        pltpu.sync_copy(ei_hbm.at[0, pl.ds(qbase, NQ)], sidx_v)
        pltpu.sync_copy(ei_hbm.at[1, pl.ds(qbase, NQ)], didx_v)

        def _slots(g, carry2):
            for b in range(NBUF):
                t = g * NBUF + b
                bs = (b + NBUF - D) % NBUF

                @pl.when(t >= NBUF)
                def _():
                    pltpu.make_async_copy(
                        rowbufs[b], acc_sh.at[didx_v.at[t - NBUF]],
                        ssems[b]).wait()

                @pl.when(t < NQ)
                def _():
                    pltpu.async_copy(
                        tab_sh.at[sidx_v.at[t]], rowbufs[b], gsems[b])

                @pl.when(jnp.logical_and(t >= D, t < NQ + D))
                def _():
                    pltpu.make_async_copy(
                        tab_sh.at[sidx_v.at[t - D]], rowbufs[bs],
                        gsems[bs]).wait()
                    pltpu.async_copy(
                        rowbufs[bs], acc_sh.at[didx_v.at[t - D]],
                        ssems[bs], add=True)
            return carry2
        return lax.fori_loop(0, (NQ + NBUF) // NBUF, _slots, carry)
    lax.fori_loop(0, 4, _quarter, 0)
    plsc.subcore_barrier()

    pltpu.sync_copy(acc_sh.at[pl.ds(r0, N // NS)],
                    out_hbm.at[pl.ds(r0, N // NS), pl.ds(c * H, H)])


_scatter_call = pl.kernel(
    _scatter_body,
    out_type=jax.ShapeDtypeStruct((N, F), jnp.float32),
    mesh=_mesh,
    compiler_params=pltpu.CompilerParams(use_tc_tiling_on_sc=False),
    scratch_types=[
        pltpu.VMEM_SHARED((ACC_ROWS, H), jnp.float32),
        pltpu.VMEM_SHARED((ACC_ROWS, H), jnp.float32),
        pltpu.VMEM((NCH // 4, CH), jnp.int32),
        pltpu.VMEM((NCH // 4, CH), jnp.int32),
        pltpu.VMEM((CH, H), jnp.float32),
        pltpu.VMEM((CH, H), jnp.float32),
        pltpu.VMEM((CH, H), jnp.float32),
        pltpu.VMEM((CH, H), jnp.float32),
        pltpu.SemaphoreType.DMA,
        pltpu.SemaphoreType.DMA,
        pltpu.SemaphoreType.DMA,
        pltpu.SemaphoreType.DMA,
        pltpu.SemaphoreType.DMA,
        pltpu.SemaphoreType.DMA,
        pltpu.SemaphoreType.DMA,
        pltpu.SemaphoreType.DMA,
    ],
)


def _dis_body(d0_ref, d1_ref, dis_ref):
    deg = d0_ref[...] + d1_ref[...] + 1.0
    dis_ref[...] = lax.rsqrt(deg)


def _mm_scale_body(x_ref, wt_ref, dis_ref, o_ref):
    h = jnp.dot(x_ref[...], wt_ref[...], preferred_element_type=jnp.float32)
    o_ref[...] = h * dis_ref[...]


def _layer2_body(a_ref, dis_ref, b_ref, wt_ref, o_ref):
    z = jnp.maximum(a_ref[...] * dis_ref[...] + b_ref[...], 0.0)
    h = jnp.dot(z, wt_ref[...], preferred_element_type=jnp.float32)
    o_ref[...] = h * dis_ref[...]


def _final_body(a_ref, dis_ref, b_ref, wt_ref, bfc_ref, o_ref):
    z = jnp.maximum(a_ref[...] * dis_ref[...] + b_ref[...], 0.0)
    m = jnp.sum(z, axis=0, keepdims=True) * (1.0 / N)
    o_ref[...] = jnp.dot(m, wt_ref[...],
                         preferred_element_type=jnp.float32) + bfc_ref[...]


_dis_call = pl.pallas_call(
    _dis_body,
    out_shape=jax.ShapeDtypeStruct((DEG_OUT // 128, 128), jnp.float32),
)

_mm_call = pl.pallas_call(
    _mm_scale_body,
    out_shape=jax.ShapeDtypeStruct((N, F), jnp.float32),
)

_layer2_call = pl.pallas_call(
    _layer2_body,
    out_shape=jax.ShapeDtypeStruct((N, F), jnp.float32),
)

_final_call = pl.pallas_call(
    _final_body,
    out_shape=jax.ShapeDtypeStruct((1, F), jnp.float32),
)


def kernel(x, edge_index, W1, b1, W2, b2, Wfc, bfc):
    ei3 = jnp.pad(edge_index.astype(jnp.int32), ((0, 0), (0, EPAD - E)),
                  constant_values=TRASH).reshape(2, EPAD // CH, CH)

    d0, d1 = _deg_call(ei3)
    dis = _dis_call(d0.reshape(DEG_OUT // 128, 128),
                    d1.reshape(DEG_OUT // 128, 128))
    dis_col = dis.reshape(DEG_OUT)[:N].reshape(N, 1)
    h1 = _mm_call(x, W1.T, dis_col)
    a1 = _scatter_call(h1, ei3)
    h2 = _layer2_call(a1, dis_col, b1.reshape(1, F), W2.T)
    a2 = _scatter_call(h2, ei3)
    out = _final_call(a2, dis_col, b2.reshape(1, F),
                      Wfc.T, bfc.reshape(1, F))
    return out


# speedup vs baseline: 39.5074x; 1.2531x over previous
import jax
import jax.numpy as jnp
from jax import lax
from jax.experimental import pallas as pl
from jax.experimental.pallas import tpu as pltpu
from jax.experimental.pallas import tpu_sc as plsc

N = 10000
E = 320000
F = 128
H = 64
NC, NS, L = 2, 16, 16
CH = 128
EPAD = 327680
NCH = EPAD // NS // CH
NBUF = 4
TRASH = N
ACC_ROWS = 10240
DEG_W = 80896
DEG_OUT = 10112

_mesh = plsc.VectorSubcoreMesh(
    core_axis_name="c", subcore_axis_name="s", num_cores=NC, num_subcores=NS)


def _deg_body(ei_hbm, out0_hbm, out1_hbm, deg_sh, zb_v, ones_v, dstb_v, dsem):
    c = lax.axis_index("c")
    s = lax.axis_index("s")
    w = c * NS + s
    ROWS = EPAD // CH // (NC * NS)
    ZQ = DEG_OUT // NS

    def _z(k, carry):
        zb_v[pl.ds(k * L, L)] = jnp.zeros((L,), jnp.float32)
        return carry
    lax.fori_loop(0, 640 // L, _z, 0)

    def _ob(k, carry):
        ones_v[pl.ds(k * L, L)] = jnp.ones((L,), jnp.float32)
        return carry
    lax.fori_loop(0, CH // L, _ob, 0)
    pltpu.sync_copy(zb_v.at[pl.ds(0, ZQ)], deg_sh.at[pl.ds(s * ZQ, ZQ)])
    pltpu.sync_copy(ei_hbm.at[1, pl.ds(w * ROWS, ROWS)], dstb_v)
    plsc.subcore_barrier()

    def _grp(g, carry):
        for b in range(8):
            pltpu.async_copy(
                ones_v, deg_sh.at[dstb_v.at[g * 8 + b]], dsem, add=True)
        for b in range(8):
            pltpu.make_async_copy(
                ones_v, deg_sh.at[dstb_v.at[g * 8 + b]], dsem).wait()
        return carry
    lax.fori_loop(0, ROWS // 8, _grp, 0)
    plsc.subcore_barrier()

    @pl.when(s == 0)
    def _():
        @pl.when(c == 0)
        def _():
            pltpu.sync_copy(deg_sh, out0_hbm)

        @pl.when(c == 1)
        def _():
            pltpu.sync_copy(deg_sh, out1_hbm)


_deg_call = pl.kernel(
    _deg_body,
    out_type=(jax.ShapeDtypeStruct((DEG_OUT,), jnp.float32),
              jax.ShapeDtypeStruct((DEG_OUT,), jnp.float32)),
    mesh=_mesh,
    compiler_params=pltpu.CompilerParams(use_tc_tiling_on_sc=False),
    scratch_types=[
        pltpu.VMEM_SHARED((DEG_OUT,), jnp.float32),
        pltpu.VMEM((640,), jnp.float32),
        pltpu.VMEM((CH,), jnp.float32),
        pltpu.VMEM((EPAD // CH // (NC * NS), CH), jnp.int32),
        pltpu.SemaphoreType.DMA,
    ],
)


def _scatter_body(h_hbm, ei_hbm, out_hbm,
                  acc_sh, tab_sh, sidx_v, didx_v,
                  rows0_v, rows1_v, rows2_v, rows3_v,
                  g0, g1, g2, g3, s0, s1, s2, s3):
    c = lax.axis_index("c")
    s = lax.axis_index("s")
    rowbufs = (rows0_v, rows1_v, rows2_v, rows3_v)
    gsems = (g0, g1, g2, g3)
    ssems = (s0, s1, s2, s3)

    r0 = s * (N // NS)
    pltpu.sync_copy(h_hbm.at[pl.ds(r0, N // NS), pl.ds(c * H, H)],
                    tab_sh.at[pl.ds(r0, N // NS)])
    pltpu.sync_copy(h_hbm.at[pl.ds(r0, N // NS), pl.ds(c * H, H)],
                    acc_sh.at[pl.ds(r0, N // NS)])
    plsc.subcore_barrier()

    NQ = NCH // 4

    D = 2

    def _quarter(q, carry):
        qbase = s * NCH + q * NQ
        pltpu.sync_copy(ei_hbm.at[0, pl.ds(qbase, NQ)], sidx_v)
        pltpu.sync_copy(ei_hbm.at[1, pl.ds(qbase, NQ)], didx_v)

        def _slots(g, carry2):
            for b in range(NBUF):
                t = g * NBUF + b
                bs = (b + NBUF - D) % NBUF

                @pl.when(t >= NBUF)
                def _():
                    pltpu.make_async_copy(
                        rowbufs[b], acc_sh.at[didx_v.at[t - NBUF]],
                        ssems[b]).wait()

                @pl.when(t < NQ)
                def _():
                    pltpu.async_copy(
                        tab_sh.at[sidx_v.at[t]], rowbufs[b], gsems[b])

                @pl.when(jnp.logical_and(t >= D, t < NQ + D))
                def _():
                    pltpu.make_async_copy(
                        tab_sh.at[sidx_v.at[t - D]], rowbufs[bs],
                        gsems[bs]).wait()
                    pltpu.async_copy(
                        rowbufs[bs], acc_sh.at[didx_v.at[t - D]],
                        ssems[bs], add=True)
            return carry2
        return lax.fori_loop(0, (NQ + NBUF) // NBUF, _slots, carry)
    lax.fori_loop(0, 4, _quarter, 0)
    plsc.subcore_barrier()

    pltpu.sync_copy(acc_sh.at[pl.ds(r0, N // NS)],
                    out_hbm.at[pl.ds(r0, N // NS), pl.ds(c * H, H)])


_scatter_call = pl.kernel(
    _scatter_body,
    out_type=jax.ShapeDtypeStruct((N, F), jnp.bfloat16),
    mesh=_mesh,
    compiler_params=pltpu.CompilerParams(use_tc_tiling_on_sc=False),
    scratch_types=[
        pltpu.VMEM_SHARED((ACC_ROWS, H), jnp.bfloat16),
        pltpu.VMEM_SHARED((ACC_ROWS, H), jnp.bfloat16),
        pltpu.VMEM((NCH // 4, CH), jnp.int32),
        pltpu.VMEM((NCH // 4, CH), jnp.int32),
        pltpu.VMEM((CH, H), jnp.bfloat16),
        pltpu.VMEM((CH, H), jnp.bfloat16),
        pltpu.VMEM((CH, H), jnp.bfloat16),
        pltpu.VMEM((CH, H), jnp.bfloat16),
        pltpu.SemaphoreType.DMA,
        pltpu.SemaphoreType.DMA,
        pltpu.SemaphoreType.DMA,
        pltpu.SemaphoreType.DMA,
        pltpu.SemaphoreType.DMA,
        pltpu.SemaphoreType.DMA,
        pltpu.SemaphoreType.DMA,
        pltpu.SemaphoreType.DMA,
    ],
)


def _dis_body(d0_ref, d1_ref, dis_ref):
    deg = d0_ref[...] + d1_ref[...] + 1.0
    dis_ref[...] = lax.rsqrt(deg)


def _mm_scale_body(x_ref, wt_ref, dis_ref, o_ref):
    h = jnp.dot(x_ref[...], wt_ref[...], preferred_element_type=jnp.float32)
    o_ref[...] = (h * dis_ref[...]).astype(jnp.bfloat16)


def _layer2_body(a_ref, dis_ref, b_ref, wt_ref, o_ref):
    a = a_ref[...].astype(jnp.float32)
    z = jnp.maximum(a * dis_ref[...] + b_ref[...], 0.0)
    h = jnp.dot(z, wt_ref[...], preferred_element_type=jnp.float32)
    o_ref[...] = (h * dis_ref[...]).astype(jnp.bfloat16)


def _final_body(a_ref, dis_ref, b_ref, wt_ref, bfc_ref, o_ref):
    a = a_ref[...].astype(jnp.float32)
    z = jnp.maximum(a * dis_ref[...] + b_ref[...], 0.0)
    m = jnp.sum(z, axis=0, keepdims=True) * (1.0 / N)
    o_ref[...] = jnp.dot(m, wt_ref[...],
                         preferred_element_type=jnp.float32) + bfc_ref[...]


_dis_call = pl.pallas_call(
    _dis_body,
    out_shape=jax.ShapeDtypeStruct((DEG_OUT // 128, 128), jnp.float32),
)

_mm_call = pl.pallas_call(
    _mm_scale_body,
    out_shape=jax.ShapeDtypeStruct((N, F), jnp.bfloat16),
)

_layer2_call = pl.pallas_call(
    _layer2_body,
    out_shape=jax.ShapeDtypeStruct((N, F), jnp.bfloat16),
)

_final_call = pl.pallas_call(
    _final_body,
    out_shape=jax.ShapeDtypeStruct((1, F), jnp.float32),
)


def kernel(x, edge_index, W1, b1, W2, b2, Wfc, bfc):
    ei3 = jnp.pad(edge_index.astype(jnp.int32), ((0, 0), (0, EPAD - E)),
                  constant_values=TRASH).reshape(2, EPAD // CH, CH)

    d0, d1 = _deg_call(ei3)
    dis = _dis_call(d0.reshape(DEG_OUT // 128, 128),
                    d1.reshape(DEG_OUT // 128, 128))
    dis_col = dis.reshape(DEG_OUT)[:N].reshape(N, 1)
    h1 = _mm_call(x, W1.T, dis_col)
    a1 = _scatter_call(h1, ei3)
    h2 = _layer2_call(a1, dis_col, b1.reshape(1, F), W2.T)
    a2 = _scatter_call(h2, ei3)
    out = _final_call(a2, dis_col, b2.reshape(1, F),
                      Wfc.T, bfc.reshape(1, F))
    return out
